# baseline (speedup 1.0000x reference)
import jax
import jax.numpy as jnp
from jax import lax
from jax.experimental import pallas as pl
from jax.experimental.pallas import tpu as pltpu

_TOKENS = 16384
_D = 512
_B = 16
_BLK = 1024
_NBLK = _TOKENS // _BLK


def _onehot(splits_row, j, blk, nseg):
    rows = lax.broadcasted_iota(jnp.int32, (blk, nseg), 0) + j * blk
    lower = jnp.broadcast_to(splits_row[:, :nseg], (blk, nseg))
    upper = jnp.broadcast_to(splits_row[:, 1:], (blk, nseg))
    return ((rows >= lower) & (rows < upper)).astype(jnp.float32)


def _body(splits_ref, x_ref, out_ref, acc_ref, xcopy_ref):
    phase = pl.program_id(0)
    j = pl.program_id(1)
    splits_row = splits_ref[:]

    @pl.when(jnp.logical_and(phase == 0, j == 0))
    def _init():
        acc_ref[:] = jnp.zeros_like(acc_ref)

    @pl.when(phase == 0)
    def _phase0():
        x_blk = x_ref[:]
        oneh = _onehot(splits_row, j, _BLK, _B)
        acc_ref[:] += lax.dot_general(
            oneh, x_blk,
            dimension_numbers=(((0,), (0,)), ((), ())),
            preferred_element_type=jnp.float32,
        )
        xcopy_ref[pl.ds(j * _BLK, _BLK), :] = x_blk

    @pl.when(jnp.logical_and(phase == 1, j == 0))
    def _finalize():
        counts = (splits_row[0, 1:] - splits_row[0, :_B]).astype(jnp.float32)
        denom = jnp.maximum(counts, 1.0)[:, None]
        acc_ref[:] = acc_ref[:] / denom

    @pl.when(phase == 1)
    def _phase1():
        oneh = _onehot(splits_row, j, _BLK, _B)
        out_ref[:, :_D] = lax.dot_general(
            oneh, acc_ref[:],
            dimension_numbers=(((1,), (0,)), ((), ())),
            preferred_element_type=jnp.float32,
        )
        out_ref[:, _D:] = xcopy_ref[pl.ds(j * _BLK, _BLK), :]


def kernel(x_data, x_row_splits):
    splits = x_row_splits.astype(jnp.int32).reshape(1, _B + 1)
    grid = (2, _NBLK)
    return pl.pallas_call(
        _body,
        grid=grid,
        in_specs=[
            pl.BlockSpec((1, _B + 1), lambda p, j: (0, 0)),
            pl.BlockSpec((_BLK, _D), lambda p, j: (jnp.where(p == 0, j, 0), 0)),
        ],
        out_specs=pl.BlockSpec((_BLK, 2 * _D), lambda p, j: (jnp.where(p == 1, j, 0), 0)),
        out_shape=jax.ShapeDtypeStruct((_TOKENS, 2 * _D), jnp.float32),
        scratch_shapes=[
            pltpu.VMEM((_B, _D), jnp.float32),
            pltpu.VMEM((_TOKENS, _D), jnp.float32),
        ],
    )(splits, x_data)
